# baseline (speedup 1.0000x reference)
import jax
import jax.numpy as jnp
from jax import lax
from jax.experimental import pallas as pl
from jax.experimental.pallas import tpu as pltpu
from jax.experimental.pallas import tpu_sc as plsc

_B, _T, _D = 16, 576, 256
_N = _B * _T
_K = 1024
_H = 2
_NH = _N // _H
_BLK = 512
_GRID = _NH // _BLK
_CBLK = 2304
_CGRID = _NH // _CBLK
_COMMIT = 0.25


def _distmin(r, e_ref, e2_ref):
    r2 = jnp.sum(r * r, axis=1, keepdims=True)
    re = jax.lax.dot_general(
        r, e_ref[...], (((1,), (1,)), ((), ())),
        preferred_element_type=jnp.float32)
    dist = r2 + e2_ref[...] - 2.0 * re
    iota = jax.lax.broadcasted_iota(jnp.int32, (_BLK, _K), 1)
    m = jnp.min(dist, axis=1, keepdims=True)
    idx = jnp.min(jnp.where(dist == m, iota, _K), axis=1).astype(jnp.int32)
    return idx, m


def _argmin0_body(x_ref, e_ref, e2_ref, i_ref):
    idx, _ = _distmin(x_ref[...], e_ref, e2_ref)
    i_ref[0, 0, :] = idx


def _argmin1_body(x_ref, q_ref, e_ref, e2_ref, i_ref, loss_ref):
    x = x_ref[...]
    q0 = q_ref[...]
    r = x - q0
    idx, m = _distmin(r, e_ref, e2_ref)
    i_ref[0, 0, :] = idx
    part = ((jnp.sum((q0 - x) ** 2) + jnp.sum(m))
            * ((1.0 + _COMMIT) / (_N * _D))).reshape(1, 1)

    @pl.when(pl.program_id(0) == 0)
    def _init():
        loss_ref[...] = jnp.zeros_like(loss_ref)

    loss_ref[...] += part


def _argmin0_call(xf, cb, e2):
    return pl.pallas_call(
        _argmin0_body,
        grid=(_GRID,),
        in_specs=[pl.BlockSpec((_BLK, _D), lambda i: (i, 0)),
                  pl.BlockSpec((_K, _D), lambda i: (0, 0)),
                  pl.BlockSpec((1, _K), lambda i: (0, 0))],
        out_specs=pl.BlockSpec((1, 1, _BLK), lambda i: (i, 0, 0)),
        out_shape=jax.ShapeDtypeStruct((_GRID, 1, _BLK), jnp.int32),
    )(xf, cb, e2)


def _argmin1_call(xf, q0, cb, e2):
    return pl.pallas_call(
        _argmin1_body,
        grid=(_GRID,),
        in_specs=[pl.BlockSpec((_BLK, _D), lambda i: (i, 0)),
                  pl.BlockSpec((_BLK, _D), lambda i: (i, 0)),
                  pl.BlockSpec((_K, _D), lambda i: (0, 0)),
                  pl.BlockSpec((1, _K), lambda i: (0, 0))],
        out_specs=[pl.BlockSpec((1, 1, _BLK), lambda i: (i, 0, 0)),
                   pl.BlockSpec((1, 1), lambda i: (0, 0))],
        out_shape=[jax.ShapeDtypeStruct((_GRID, 1, _BLK), jnp.int32),
                   jax.ShapeDtypeStruct((1, 1), jnp.float32)],
    )(xf, q0, cb, e2)


_SC_INFO = plsc.get_sparse_core_info()
_NC = _SC_INFO.num_cores
_NS = _SC_INFO.num_subcores
_NW = _NC * _NS
_ROWS_PER_W = _NH // _NW


def _gather_body(table_hbm, idx_hbm, out_hbm, idx_v, rows_v, sem):
    wid = lax.axis_index("s") * _NC + lax.axis_index("c")
    base = wid * _ROWS_PER_W
    pltpu.sync_copy(idx_hbm.at[pl.ds(base, _ROWS_PER_W)], idx_v)
    pltpu.async_copy(table_hbm.at[idx_v], rows_v, sem).wait()
    pltpu.sync_copy(rows_v, out_hbm.at[pl.ds(base, _ROWS_PER_W)])


_sc_gather = pl.kernel(
    _gather_body,
    mesh=plsc.VectorSubcoreMesh(core_axis_name="c", subcore_axis_name="s"),
    out_type=jax.ShapeDtypeStruct((_NH, _D), jnp.float32),
    scratch_types=[
        pltpu.VMEM((_ROWS_PER_W,), jnp.int32),
        pltpu.VMEM((_ROWS_PER_W, _D), jnp.float32),
        pltpu.SemaphoreType.DMA,
    ],
)


def _combine_body(x_ref, q0_ref, q1_ref, out_ref):
    x = x_ref[...]
    out_ref[...] = x + ((q0_ref[...] + q1_ref[...]) - x)


def _combine_call(xf, q0, q1):
    return pl.pallas_call(
        _combine_body,
        grid=(_CGRID,),
        in_specs=[pl.BlockSpec((_CBLK, _D), lambda i: (i, 0))] * 3,
        out_specs=pl.BlockSpec((_CBLK, _D), lambda i: (i, 0)),
        out_shape=jax.ShapeDtypeStruct((_NH, _D), jnp.float32),
    )(xf, q0, q1)


def kernel(x, cb0, cb1):
    b, t, d = x.shape
    xf = x.reshape(b * t, d)
    e20 = (cb0 ** 2).sum(axis=1)[None, :]
    e21 = (cb1 ** 2).sum(axis=1)[None, :]

    quant_h, i0_h, i1_h, loss_h = [], [], [], []
    for h in range(_H):
        xh = lax.slice_in_dim(xf, h * _NH, (h + 1) * _NH, axis=0)
        i0 = _argmin0_call(xh, cb0, e20)
        q0 = _sc_gather(cb0, i0.reshape(_NH))
        i1, loss = _argmin1_call(xh, q0, cb1, e21)
        q1 = _sc_gather(cb1, i1.reshape(_NH))
        quant_h.append(_combine_call(xh, q0, q1))
        i0_h.append(i0)
        i1_h.append(i1)
        loss_h.append(loss[0, 0])

    quantized = jnp.concatenate(quant_h, axis=0).reshape(b, t, d)
    codes = jnp.stack(
        [jnp.concatenate([i.reshape(_NH) for i in i0_h]).reshape(b, t),
         jnp.concatenate([i.reshape(_NH) for i in i1_h]).reshape(b, t)],
        axis=0)
    return quantized, codes, loss_h[0] + loss_h[1]


# speedup vs baseline: 1.2107x; 1.2107x over previous
import jax
import jax.numpy as jnp
from jax import lax
from jax.experimental import pallas as pl
from jax.experimental.pallas import tpu as pltpu
from jax.experimental.pallas import tpu_sc as plsc

_B, _T, _D = 16, 576, 256
_N = _B * _T
_K = 1024
_H = 1
_NH = _N // _H
_BLK = 1024
_GRID = _NH // _BLK
_CBLK = 3072
_CGRID = _NH // _CBLK
_COMMIT = 0.25


def _distmin(r, e_ref, e2_ref):
    r2 = jnp.sum(r * r, axis=1, keepdims=True)
    re = jax.lax.dot_general(
        r, e_ref[...], (((1,), (1,)), ((), ())),
        preferred_element_type=jnp.float32)
    dist = r2 + e2_ref[...] - 2.0 * re
    iota = jax.lax.broadcasted_iota(jnp.int32, (_BLK, _K), 1)
    m = jnp.min(dist, axis=1, keepdims=True)
    idx = jnp.min(jnp.where(dist == m, iota, _K), axis=1).astype(jnp.int32)
    return idx, m


def _argmin0_body(x_ref, e_ref, e2_ref, i_ref):
    idx, _ = _distmin(x_ref[...], e_ref, e2_ref)
    i_ref[0, 0, :] = idx


def _argmin1_body(x_ref, q_ref, e_ref, e2_ref, i_ref, loss_ref):
    x = x_ref[...]
    q0 = q_ref[...]
    r = x - q0
    idx, m = _distmin(r, e_ref, e2_ref)
    i_ref[0, 0, :] = idx
    part = ((jnp.sum((q0 - x) ** 2) + jnp.sum(m))
            * ((1.0 + _COMMIT) / (_N * _D))).reshape(1, 1)

    @pl.when(pl.program_id(0) == 0)
    def _init():
        loss_ref[...] = jnp.zeros_like(loss_ref)

    loss_ref[...] += part


def _argmin0_call(xf, cb, e2):
    return pl.pallas_call(
        _argmin0_body,
        grid=(_GRID,),
        in_specs=[pl.BlockSpec((_BLK, _D), lambda i: (i, 0)),
                  pl.BlockSpec((_K, _D), lambda i: (0, 0)),
                  pl.BlockSpec((1, _K), lambda i: (0, 0))],
        out_specs=pl.BlockSpec((1, 1, _BLK), lambda i: (i, 0, 0)),
        out_shape=jax.ShapeDtypeStruct((_GRID, 1, _BLK), jnp.int32),
    )(xf, cb, e2)


def _argmin1_call(xf, q0, cb, e2):
    return pl.pallas_call(
        _argmin1_body,
        grid=(_GRID,),
        in_specs=[pl.BlockSpec((_BLK, _D), lambda i: (i, 0)),
                  pl.BlockSpec((_BLK, _D), lambda i: (i, 0)),
                  pl.BlockSpec((_K, _D), lambda i: (0, 0)),
                  pl.BlockSpec((1, _K), lambda i: (0, 0))],
        out_specs=[pl.BlockSpec((1, 1, _BLK), lambda i: (i, 0, 0)),
                   pl.BlockSpec((1, 1), lambda i: (0, 0))],
        out_shape=[jax.ShapeDtypeStruct((_GRID, 1, _BLK), jnp.int32),
                   jax.ShapeDtypeStruct((1, 1), jnp.float32)],
    )(xf, q0, cb, e2)


_SC_INFO = plsc.get_sparse_core_info()
_NC = _SC_INFO.num_cores
_NS = _SC_INFO.num_subcores
_NW = _NC * _NS
_ROWS_PER_W = _NH // _NW


def _gather_body(table_hbm, idx_hbm, out_hbm, idx_v, rows_v, sem):
    wid = lax.axis_index("s") * _NC + lax.axis_index("c")
    base = wid * _ROWS_PER_W
    pltpu.sync_copy(idx_hbm.at[pl.ds(base, _ROWS_PER_W)], idx_v)
    pltpu.async_copy(table_hbm.at[idx_v], rows_v, sem).wait()
    pltpu.sync_copy(rows_v, out_hbm.at[pl.ds(base, _ROWS_PER_W)])


_sc_gather = pl.kernel(
    _gather_body,
    mesh=plsc.VectorSubcoreMesh(core_axis_name="c", subcore_axis_name="s"),
    out_type=jax.ShapeDtypeStruct((_NH, _D), jnp.float32),
    scratch_types=[
        pltpu.VMEM((_ROWS_PER_W,), jnp.int32),
        pltpu.VMEM((_ROWS_PER_W, _D), jnp.float32),
        pltpu.SemaphoreType.DMA,
    ],
)


def _combine_body(x_ref, q0_ref, q1_ref, out_ref):
    x = x_ref[...]
    out_ref[...] = x + ((q0_ref[...] + q1_ref[...]) - x)


def _combine_call(xf, q0, q1):
    return pl.pallas_call(
        _combine_body,
        grid=(_CGRID,),
        in_specs=[pl.BlockSpec((_CBLK, _D), lambda i: (i, 0))] * 3,
        out_specs=pl.BlockSpec((_CBLK, _D), lambda i: (i, 0)),
        out_shape=jax.ShapeDtypeStruct((_NH, _D), jnp.float32),
    )(xf, q0, q1)


def kernel(x, cb0, cb1):
    b, t, d = x.shape
    xf = x.reshape(b * t, d)
    e20 = (cb0 ** 2).sum(axis=1)[None, :]
    e21 = (cb1 ** 2).sum(axis=1)[None, :]

    quant_h, i0_h, i1_h, loss_h = [], [], [], []
    for h in range(_H):
        xh = lax.slice_in_dim(xf, h * _NH, (h + 1) * _NH, axis=0)
        i0 = _argmin0_call(xh, cb0, e20)
        q0 = _sc_gather(cb0, i0.reshape(_NH))
        i1, loss = _argmin1_call(xh, q0, cb1, e21)
        q1 = _sc_gather(cb1, i1.reshape(_NH))
        quant_h.append(_combine_call(xh, q0, q1))
        i0_h.append(i0)
        i1_h.append(i1)
        loss_h.append(loss[0, 0])

    quantized = jnp.concatenate(quant_h, axis=0).reshape(b, t, d)
    codes = jnp.stack(
        [jnp.concatenate([i.reshape(_NH) for i in i0_h]).reshape(b, t),
         jnp.concatenate([i.reshape(_NH) for i in i1_h]).reshape(b, t)],
        axis=0)
    return quantized, codes, loss_h[0]
